# baseline (speedup 1.0000x reference)
import jax
import jax.numpy as jnp
from jax import lax
from jax.experimental import pallas as pl
from jax.experimental.pallas import tpu as pltpu
from jax.experimental.pallas import tpu_sc as plsc

N_FIELDS = 26
BATCH = 4096
VOCAB = 100000
EMBED_DIM = 32
LANES = 16
N_WORKERS = 32
B_PER_W = BATCH // N_WORKERS
ROWS_PER_W = B_PER_W * N_FIELDS


def _concat_embed_body(x_hbm, tbl_hbm, out_hbm, x_vm, idx_vm, rows_vm, sem):
    wid = lax.axis_index("s") * 2 + lax.axis_index("c")
    base = wid * B_PER_W
    obase = wid * ROWS_PER_W

    pltpu.sync_copy(x_hbm.at[:, pl.ds(base, B_PER_W)], x_vm)

    lanes = lax.iota(jnp.int32, 16)
    for i in range(N_FIELDS):
        for v in range(B_PER_W // LANES):
            vals = x_vm[i, pl.ds(v * LANES, LANES)] + i * VOCAB
            pos = (lanes + v * LANES) * N_FIELDS + i
            plsc.store_scatter(idx_vm, [pos], vals)

    for s in range(N_FIELDS):
        idx_sl = idx_vm.at[pl.ds(s * B_PER_W, B_PER_W)]
        pltpu.async_copy(tbl_hbm.at[idx_sl], rows_vm, sem).wait()
        pltpu.sync_copy(rows_vm,
                        out_hbm.at[pl.ds(obase + s * B_PER_W, B_PER_W)])


def kernel(x, tables):
    tbl_flat = tables.reshape(N_FIELDS * VOCAB, EMBED_DIM)
    mesh = plsc.VectorSubcoreMesh(core_axis_name="c", subcore_axis_name="s")
    k = pl.kernel(
        _concat_embed_body,
        mesh=mesh,
        out_type=jax.ShapeDtypeStruct((BATCH * N_FIELDS, EMBED_DIM),
                                      jnp.float32),
        scratch_types=[
            pltpu.VMEM((N_FIELDS, B_PER_W), jnp.int32),
            pltpu.VMEM((ROWS_PER_W,), jnp.int32),
            pltpu.VMEM((B_PER_W, EMBED_DIM), jnp.float32),
            pltpu.SemaphoreType.DMA,
        ],
        compiler_params=pltpu.CompilerParams(needs_layout_passes=False,
                                             use_tc_tiling_on_sc=False),
    )
    out_flat = k(x, tbl_flat)
    return out_flat.reshape(BATCH, N_FIELDS * EMBED_DIM)


# speedup vs baseline: 1.0158x; 1.0158x over previous
import jax
import jax.numpy as jnp
from jax import lax
from jax.experimental import pallas as pl
from jax.experimental.pallas import tpu as pltpu
from jax.experimental.pallas import tpu_sc as plsc

N_FIELDS = 26
BATCH = 4096
VOCAB = 100000
EMBED_DIM = 32
LANES = 16
N_WORKERS = 32
B_PER_W = BATCH // N_WORKERS
ROWS_PER_W = B_PER_W * N_FIELDS


def _concat_embed_body(x_hbm, tbl_hbm, out_hbm, x_vm, idx_vm, rows_vm, sem):
    wid = lax.axis_index("s") * 2 + lax.axis_index("c")
    base = wid * B_PER_W
    obase = wid * ROWS_PER_W

    pltpu.sync_copy(x_hbm.at[:, pl.ds(base, B_PER_W)], x_vm)

    lanes = lax.iota(jnp.int32, 16)
    for i in range(N_FIELDS):
        for v in range(B_PER_W // LANES):
            vals = x_vm[i, pl.ds(v * LANES, LANES)] + i * VOCAB
            pos = (lanes + v * LANES) * N_FIELDS + i
            plsc.store_scatter(idx_vm, [pos], vals)

    copies = []
    for s in range(N_FIELDS):
        idx_sl = idx_vm.at[pl.ds(s * B_PER_W, B_PER_W)]
        dst = rows_vm.at[pl.ds(s * B_PER_W, B_PER_W)]
        copies.append(pltpu.async_copy(tbl_hbm.at[idx_sl], dst, sem))
    for c in copies:
        c.wait()
    pltpu.sync_copy(rows_vm, out_hbm.at[pl.ds(obase, ROWS_PER_W)])


def kernel(x, tables):
    tbl_flat = tables.reshape(N_FIELDS * VOCAB, EMBED_DIM)
    mesh = plsc.VectorSubcoreMesh(core_axis_name="c", subcore_axis_name="s")
    k = pl.kernel(
        _concat_embed_body,
        mesh=mesh,
        out_type=jax.ShapeDtypeStruct((BATCH * N_FIELDS, EMBED_DIM),
                                      jnp.float32),
        scratch_types=[
            pltpu.VMEM((N_FIELDS, B_PER_W), jnp.int32),
            pltpu.VMEM((ROWS_PER_W,), jnp.int32),
            pltpu.VMEM((ROWS_PER_W, EMBED_DIM), jnp.float32),
            pltpu.SemaphoreType.DMA,
        ],
        compiler_params=pltpu.CompilerParams(needs_layout_passes=False,
                                             use_tc_tiling_on_sc=False),
    )
    out_flat = k(x, tbl_flat)
    return out_flat.reshape(BATCH, N_FIELDS * EMBED_DIM)


# speedup vs baseline: 6.2184x; 6.1219x over previous
import jax
import jax.numpy as jnp
from jax import lax
from jax.experimental import pallas as pl
from jax.experimental.pallas import tpu as pltpu
from jax.experimental.pallas import tpu_sc as plsc

N_FIELDS = 26
BATCH = 4096
VOCAB = 100000
EMBED_DIM = 32
LANES = 16
N_WORKERS = 32
N_FEATURES = N_FIELDS * EMBED_DIM
ROWS_PER_W = N_FEATURES // N_WORKERS


def _concat_embed_body(x_hbm, tbl_hbm, out_hbm, x_vm, row_vm, orow_vm, sem):
    wid = lax.axis_index("s") * 2 + lax.axis_index("c")

    def one_row(jj, carry):
        c = wid * ROWS_PER_W + jj
        i = c // EMBED_DIM
        d = c % EMBED_DIM
        pltpu.sync_copy(x_hbm.at[i], x_vm)
        pltpu.sync_copy(tbl_hbm.at[i, d], row_vm)
        for g in range(BATCH // LANES):
            sl = pl.ds(g * LANES, LANES)
            orow_vm[sl] = plsc.load_gather(row_vm, [x_vm[sl]])
        pltpu.sync_copy(orow_vm, out_hbm.at[c])
        return carry

    lax.fori_loop(0, ROWS_PER_W, one_row, 0)


def kernel(x, tables):
    tbl_t = jnp.transpose(tables, (0, 2, 1))
    mesh = plsc.VectorSubcoreMesh(core_axis_name="c", subcore_axis_name="s")
    k = pl.kernel(
        _concat_embed_body,
        mesh=mesh,
        out_type=jax.ShapeDtypeStruct((N_FEATURES, BATCH), jnp.float32),
        scratch_types=[
            pltpu.VMEM((BATCH,), jnp.int32),
            pltpu.VMEM((VOCAB,), jnp.float32),
            pltpu.VMEM((BATCH,), jnp.float32),
            pltpu.SemaphoreType.DMA,
        ],
        compiler_params=pltpu.CompilerParams(needs_layout_passes=False),
    )
    out_t = k(x, tbl_t)
    return jnp.transpose(out_t)
